# Initial kernel scaffold; baseline (speedup 1.0000x reference)
import jax
import jax.numpy as jnp
from jax.experimental import pallas as pl


def kernel(inputs, embed_table, fc_w, fc_b):
    raise NotImplementedError("write your pallas kernel here")


# speedup vs baseline: 1.0057x; 1.0057x over previous
import functools

import jax
import jax.numpy as jnp
from jax import lax
from jax.experimental import pallas as pl
from jax.experimental.pallas import tpu as pltpu
from jax.experimental.pallas import tpu_sc as plsc

_NUM_CORES = 2
_NUM_SUBCORES = 16
_NW = _NUM_CORES * _NUM_SUBCORES


def _make_gather_pool(B, CTX, D, b_per_w):
  mesh = plsc.VectorSubcoreMesh(
      core_axis_name="c", subcore_axis_name="s", num_cores=_NUM_CORES,
      num_subcores=_NUM_SUBCORES)

  @functools.partial(
      pl.kernel,
      mesh=mesh,
      compiler_params=pltpu.CompilerParams(use_tc_tiling_on_sc=False),
      out_type=jax.ShapeDtypeStruct((B, D), jnp.float32),
      scratch_types=[
          pltpu.VMEM((CTX, b_per_w), jnp.int32),
          pltpu.VMEM((b_per_w, D), jnp.float32),
          pltpu.SemaphoreType.DMA,
      ],
  )
  def gather_pool(idx_hbm, table_hbm, out_hbm, idx_v, acc_v, sem):
    wid = lax.axis_index("s") * _NUM_CORES + lax.axis_index("c")
    pltpu.sync_copy(idx_hbm.at[wid], idx_v)
    pltpu.async_copy(table_hbm.at[idx_v.at[0]], acc_v, sem).wait()

    def fire(j, carry):
      pltpu.async_copy(table_hbm.at[idx_v.at[j]], acc_v, sem, add=True)
      return carry

    lax.fori_loop(1, CTX, fire, 0)

    def drain(j, carry):
      pltpu.make_async_copy(table_hbm.at[idx_v.at[0]], acc_v, sem).wait()
      return carry

    lax.fori_loop(1, CTX, drain, 0)

    pltpu.sync_copy(acc_v, out_hbm.at[pl.ds(wid * b_per_w, b_per_w)])

  return gather_pool


def _linear_body(x_ref, w_ref, b_ref, o_ref):
  o_ref[...] = (
      jnp.dot(x_ref[...], w_ref[...], preferred_element_type=jnp.float32)
      + b_ref[...]
  )


def _linear(pooled, w_t, bias2d, bm):
  B, D = pooled.shape
  N = w_t.shape[1]
  return pl.pallas_call(
      _linear_body,
      grid=(B // bm,),
      in_specs=[
          pl.BlockSpec((bm, D), lambda i: (i, 0)),
          pl.BlockSpec((D, N), lambda i: (0, 0)),
          pl.BlockSpec((1, N), lambda i: (0, 0)),
      ],
      out_specs=pl.BlockSpec((bm, N), lambda i: (i, 0)),
      out_shape=jax.ShapeDtypeStruct((B, N), jnp.float32),
  )(pooled, w_t, bias2d)


def kernel(inputs, embed_table, fc_w, fc_b):
  B, CTX = inputs.shape
  V, D = embed_table.shape
  N = fc_w.shape[0]
  b_per_w = B // _NW

  idx = inputs.astype(jnp.int32).reshape(_NW, b_per_w, CTX)
  idx = jnp.transpose(idx, (0, 2, 1))

  pooled = _make_gather_pool(B, CTX, D, b_per_w)(idx, embed_table)
  return _linear(pooled, fc_w.T, fc_b.reshape(1, N), bm=512)


# speedup vs baseline: 1.0076x; 1.0019x over previous
import functools

import jax
import jax.numpy as jnp
from jax import lax
from jax.experimental import pallas as pl
from jax.experimental.pallas import tpu as pltpu
from jax.experimental.pallas import tpu_sc as plsc

_NUM_CORES = 2
_NUM_SUBCORES = 16
_NW = _NUM_CORES * _NUM_SUBCORES
_LANES = 16


def _make_gather_pool(B, CTX, D, b_per_w):
  mesh = plsc.VectorSubcoreMesh(
      core_axis_name="c", subcore_axis_name="s", num_cores=_NUM_CORES,
      num_subcores=_NUM_SUBCORES)

  @functools.partial(
      pl.kernel,
      mesh=mesh,
      compiler_params=pltpu.CompilerParams(
          use_tc_tiling_on_sc=False, needs_layout_passes=False),
      out_type=jax.ShapeDtypeStruct((B, D), jnp.float32),
      scratch_types=[
          pltpu.VMEM((b_per_w, CTX), jnp.int32),
          pltpu.VMEM((CTX, b_per_w), jnp.int32),
          pltpu.VMEM((b_per_w, D), jnp.float32),
          pltpu.SemaphoreType.DMA,
      ],
  )
  def gather_pool(idx_hbm, table_hbm, out_hbm, idx_v, idx_t, acc_v, sem):
    wid = lax.axis_index("s") * _NUM_CORES + lax.axis_index("c")
    base = wid * b_per_w
    pltpu.sync_copy(idx_hbm.at[pl.ds(base, b_per_w)], idx_v)

    lanes = lax.iota(jnp.int32, _LANES)

    def transpose_body(j, carry):
      cols = jnp.full((_LANES,), j, jnp.int32)
      for i in range(b_per_w // _LANES):
        vals = plsc.load_gather(idx_v, [lanes + i * _LANES, cols])
        idx_t[j, pl.ds(i * _LANES, _LANES)] = vals
      return carry

    lax.fori_loop(0, CTX, transpose_body, 0)

    pltpu.async_copy(table_hbm.at[idx_t.at[0]], acc_v, sem).wait()

    def fire(j, carry):
      pltpu.async_copy(table_hbm.at[idx_t.at[j]], acc_v, sem, add=True)
      return carry

    lax.fori_loop(1, CTX, fire, 0)

    def drain(j, carry):
      pltpu.make_async_copy(table_hbm.at[idx_t.at[0]], acc_v, sem).wait()
      return carry

    lax.fori_loop(1, CTX, drain, 0)

    pltpu.sync_copy(acc_v, out_hbm.at[pl.ds(base, b_per_w)])

  return gather_pool


def _linear_body(x_ref, w_ref, b_ref, o_ref):
  o_ref[...] = (
      lax.dot_general(
          x_ref[...], w_ref[...], (((1,), (1,)), ((), ())),
          preferred_element_type=jnp.float32)
      + b_ref[...]
  )


def _linear(pooled, w, bias2d, bm):
  B, D = pooled.shape
  N = w.shape[0]
  return pl.pallas_call(
      _linear_body,
      grid=(B // bm,),
      in_specs=[
          pl.BlockSpec((bm, D), lambda i: (i, 0)),
          pl.BlockSpec((N, D), lambda i: (0, 0)),
          pl.BlockSpec((1, N), lambda i: (0, 0)),
      ],
      out_specs=pl.BlockSpec((bm, N), lambda i: (i, 0)),
      out_shape=jax.ShapeDtypeStruct((B, N), jnp.float32),
  )(pooled, w, bias2d)


def kernel(inputs, embed_table, fc_w, fc_b):
  B, CTX = inputs.shape
  V, D = embed_table.shape
  N = fc_w.shape[0]
  b_per_w = B // _NW

  idx = inputs.astype(jnp.int32)
  pooled = _make_gather_pool(B, CTX, D, b_per_w)(idx, embed_table)
  return _linear(pooled, fc_w, fc_b.reshape(1, N), bm=512)


# speedup vs baseline: 1.6691x; 1.6565x over previous
import functools

import jax
import jax.numpy as jnp
from jax import lax
from jax.experimental import pallas as pl
from jax.experimental.pallas import tpu as pltpu
from jax.experimental.pallas import tpu_sc as plsc

_NUM_CORES = 2
_NUM_SUBCORES = 16
_NW = _NUM_CORES * _NUM_SUBCORES
_LANES = 16


_VB = 2048


def _tr_body(x_ref, o_ref):
  xt = x_ref[...].T
  d = xt.shape[1]
  o_ref[:, 0:d] = xt[0:_VB, :]
  o_ref[:, d : 2 * d] = xt[_VB : 2 * _VB, :]


def _transpose_detile(table):
  D, V = table.shape
  grid = -(-V // (2 * _VB))
  out = pl.pallas_call(
      _tr_body,
      grid=(grid,),
      in_specs=[pl.BlockSpec((D, 2 * _VB), lambda i: (0, i))],
      out_specs=pl.BlockSpec((_VB, 2 * D), lambda i: (i, 0)),
      out_shape=jax.ShapeDtypeStruct((grid * _VB, 2 * D), jnp.float32),
  )(table)
  return out.reshape(2 * grid * _VB, D)


def _make_gather_pool(B, CTX, D, b_per_w):
  mesh = plsc.VectorSubcoreMesh(
      core_axis_name="c", subcore_axis_name="s", num_cores=_NUM_CORES,
      num_subcores=_NUM_SUBCORES)

  @functools.partial(
      pl.kernel,
      mesh=mesh,
      compiler_params=pltpu.CompilerParams(
          use_tc_tiling_on_sc=False, needs_layout_passes=False),
      out_type=jax.ShapeDtypeStruct((B, D), jnp.float32),
      scratch_types=[
          pltpu.VMEM((b_per_w, CTX), jnp.int32),
          pltpu.VMEM((CTX, b_per_w), jnp.int32),
          pltpu.VMEM((b_per_w, D), jnp.float32),
          pltpu.SemaphoreType.DMA,
      ],
  )
  def gather_pool(idx_hbm, table_hbm, out_hbm, idx_v, idx_t, acc_v, sem):
    wid = lax.axis_index("s") * _NUM_CORES + lax.axis_index("c")
    base = wid * b_per_w
    pltpu.sync_copy(idx_hbm.at[pl.ds(base, b_per_w)], idx_v)

    lanes = lax.iota(jnp.int32, _LANES)

    def transpose_body(j, carry):
      cols = jnp.full((_LANES,), j, jnp.int32)
      for i in range(b_per_w // _LANES):
        vals = plsc.load_gather(idx_v, [lanes + i * _LANES, cols])
        idx_t[j, pl.ds(i * _LANES, _LANES)] = vals
      return carry

    lax.fori_loop(0, CTX, transpose_body, 0)

    pltpu.async_copy(table_hbm.at[idx_t.at[0]], acc_v, sem).wait()

    def fire(j, carry):
      pltpu.async_copy(table_hbm.at[idx_t.at[j]], acc_v, sem, add=True)
      return carry

    lax.fori_loop(1, CTX, fire, 0)

    def drain(j, carry):
      pltpu.make_async_copy(table_hbm.at[idx_t.at[0]], acc_v, sem).wait()
      return carry

    lax.fori_loop(1, CTX, drain, 0)

    pltpu.sync_copy(acc_v, out_hbm.at[pl.ds(base, b_per_w)])

  return gather_pool


def _linear_body(x_ref, w_ref, b_ref, o_ref):
  o_ref[...] = (
      lax.dot_general(
          x_ref[...], w_ref[...], (((1,), (1,)), ((), ())),
          preferred_element_type=jnp.float32)
      + b_ref[...]
  )


def _linear(pooled, w, bias2d, bm):
  B, D = pooled.shape
  N = w.shape[0]
  return pl.pallas_call(
      _linear_body,
      grid=(B // bm,),
      in_specs=[
          pl.BlockSpec((bm, D), lambda i: (i, 0)),
          pl.BlockSpec((N, D), lambda i: (0, 0)),
          pl.BlockSpec((1, N), lambda i: (0, 0)),
      ],
      out_specs=pl.BlockSpec((bm, N), lambda i: (i, 0)),
      out_shape=jax.ShapeDtypeStruct((B, N), jnp.float32),
  )(pooled, w, bias2d)


def kernel(inputs, embed_table, fc_w, fc_b):
  B, CTX = inputs.shape
  V, D = embed_table.shape
  N = fc_w.shape[0]
  b_per_w = B // _NW

  v = inputs.astype(jnp.int32)
  idx = 2 * ((v // (2 * _VB)) * _VB + (v % _VB)) + ((v // _VB) % 2)
  table_lin = _transpose_detile(jnp.transpose(embed_table))
  pooled = _make_gather_pool(B, CTX, D, b_per_w)(idx, table_lin)
  return _linear(pooled, fc_w, fc_b.reshape(1, N), bm=512)


# speedup vs baseline: 1.6960x; 1.0161x over previous
import functools

import jax
import jax.numpy as jnp
from jax import lax
from jax.experimental import pallas as pl
from jax.experimental.pallas import tpu as pltpu
from jax.experimental.pallas import tpu_sc as plsc

_NUM_CORES = 2
_NUM_SUBCORES = 16
_NW = _NUM_CORES * _NUM_SUBCORES
_LANES = 16


_VB = 2048


def _tr_body(x_ref, o_ref):
  xt = x_ref[...].T
  d = xt.shape[1]
  o_ref[:, 0:d] = xt[0:_VB, :]
  o_ref[:, d : 2 * d] = xt[_VB : 2 * _VB, :]


def _transpose_detile(table):
  D, V = table.shape
  grid = -(-V // (2 * _VB))
  out = pl.pallas_call(
      _tr_body,
      grid=(grid,),
      in_specs=[pl.BlockSpec((D, 2 * _VB), lambda i: (0, i))],
      out_specs=pl.BlockSpec((_VB, 2 * D), lambda i: (i, 0)),
      out_shape=jax.ShapeDtypeStruct((grid * _VB, 2 * D), jnp.float32),
  )(table)
  return out.reshape(2 * grid * _VB, D)


def _make_gather_pool(B, CTX, D, b_per_w):
  mesh = plsc.VectorSubcoreMesh(
      core_axis_name="c", subcore_axis_name="s", num_cores=_NUM_CORES,
      num_subcores=_NUM_SUBCORES)

  @functools.partial(
      pl.kernel,
      mesh=mesh,
      compiler_params=pltpu.CompilerParams(use_tc_tiling_on_sc=False),
      out_type=jax.ShapeDtypeStruct((B, D), jnp.float32),
      scratch_types=[
          pltpu.VMEM((CTX, b_per_w), jnp.int32),
          pltpu.VMEM((b_per_w, D), jnp.float32),
          pltpu.SemaphoreType.DMA,
      ],
  )
  def gather_pool(idx_hbm, table_hbm, out_hbm, idx_t, acc_v, sem):
    wid = lax.axis_index("s") * _NUM_CORES + lax.axis_index("c")
    base = wid * b_per_w
    pltpu.sync_copy(idx_hbm.at[wid], idx_t)

    pltpu.async_copy(table_hbm.at[idx_t.at[0]], acc_v, sem).wait()

    def fire(j, carry):
      pltpu.async_copy(table_hbm.at[idx_t.at[j]], acc_v, sem, add=True)
      return carry

    lax.fori_loop(1, CTX, fire, 0)

    def drain(j, carry):
      pltpu.make_async_copy(table_hbm.at[idx_t.at[0]], acc_v, sem).wait()
      return carry

    lax.fori_loop(1, CTX, drain, 0)

    pltpu.sync_copy(acc_v, out_hbm.at[pl.ds(base, b_per_w)])

  return gather_pool


def _linear_body(x_ref, w_ref, b_ref, o_ref):
  o_ref[...] = (
      lax.dot_general(
          x_ref[...], w_ref[...], (((1,), (1,)), ((), ())),
          preferred_element_type=jnp.float32)
      + b_ref[...]
  )


def _linear(pooled, w, bias2d, bm):
  B, D = pooled.shape
  N = w.shape[0]
  return pl.pallas_call(
      _linear_body,
      grid=(B // bm,),
      in_specs=[
          pl.BlockSpec((bm, D), lambda i: (i, 0)),
          pl.BlockSpec((N, D), lambda i: (0, 0)),
          pl.BlockSpec((1, N), lambda i: (0, 0)),
      ],
      out_specs=pl.BlockSpec((bm, N), lambda i: (i, 0)),
      out_shape=jax.ShapeDtypeStruct((B, N), jnp.float32),
  )(pooled, w, bias2d)


def kernel(inputs, embed_table, fc_w, fc_b):
  B, CTX = inputs.shape
  V, D = embed_table.shape
  N = fc_w.shape[0]
  b_per_w = B // _NW

  v = inputs.astype(jnp.int32)
  idx = 2 * ((v // (2 * _VB)) * _VB + (v % _VB)) + ((v // _VB) % 2)
  idx = jnp.transpose(idx.reshape(_NW, b_per_w, CTX), (0, 2, 1))
  table_lin = _transpose_detile(jnp.transpose(embed_table))
  pooled = _make_gather_pool(B, CTX, D, b_per_w)(idx, table_lin)
  return _linear(pooled, fc_w, fc_b.reshape(1, N), bm=512)


# speedup vs baseline: 2.0768x; 1.2245x over previous
import functools

import jax
import jax.numpy as jnp
from jax import lax
from jax.experimental import pallas as pl
from jax.experimental.pallas import tpu as pltpu
from jax.experimental.pallas import tpu_sc as plsc

_NUM_CORES = 2
_NUM_SUBCORES = 16
_NW = _NUM_CORES * _NUM_SUBCORES
_LANES = 16


_VB = 2048


def _tr_body(x_ref, o_ref):
  x = x_ref[...]
  xc = jnp.concatenate([x[:, :_VB], x[:, _VB:]], axis=0)
  o_ref[...] = xc.T


def _transpose_detile(table):
  D, V = table.shape
  grid = -(-V // (2 * _VB))
  out = pl.pallas_call(
      _tr_body,
      grid=(grid,),
      in_specs=[pl.BlockSpec((D, 2 * _VB), lambda i: (0, i))],
      out_specs=pl.BlockSpec((_VB, 2 * D), lambda i: (i, 0)),
      out_shape=jax.ShapeDtypeStruct((grid * _VB, 2 * D), jnp.float32),
  )(table)
  return out.reshape(2 * grid * _VB, D)


def _make_gather_pool(B, CTX, D, b_per_w):
  mesh = plsc.VectorSubcoreMesh(
      core_axis_name="c", subcore_axis_name="s", num_cores=_NUM_CORES,
      num_subcores=_NUM_SUBCORES)

  @functools.partial(
      pl.kernel,
      mesh=mesh,
      compiler_params=pltpu.CompilerParams(use_tc_tiling_on_sc=False),
      out_type=jax.ShapeDtypeStruct((B, D), jnp.float32),
      scratch_types=[
          pltpu.VMEM((CTX, b_per_w), jnp.int32),
          pltpu.VMEM((b_per_w, D), jnp.float32),
          pltpu.SemaphoreType.DMA,
      ],
  )
  def gather_pool(idx_hbm, table_hbm, out_hbm, idx_t, acc_v, sem):
    wid = lax.axis_index("s") * _NUM_CORES + lax.axis_index("c")
    base = wid * b_per_w
    pltpu.sync_copy(idx_hbm.at[wid], idx_t)

    pltpu.async_copy(table_hbm.at[idx_t.at[0]], acc_v, sem).wait()

    def fire(j, carry):
      pltpu.async_copy(table_hbm.at[idx_t.at[j]], acc_v, sem, add=True)
      return carry

    lax.fori_loop(1, CTX, fire, 0)

    def drain(j, carry):
      pltpu.make_async_copy(table_hbm.at[idx_t.at[0]], acc_v, sem).wait()
      return carry

    lax.fori_loop(1, CTX, drain, 0)

    pltpu.sync_copy(acc_v, out_hbm.at[pl.ds(base, b_per_w)])

  return gather_pool


def _linear_body(w_ref, x_ref, b_ref, o_ref):
  o_ref[...] = (
      lax.dot_general(
          w_ref[...], x_ref[...], (((1,), (1,)), ((), ())),
          preferred_element_type=jnp.float32)
      + b_ref[...]
  )


def _linear(pooled, w, bias_col, bm):
  B, D = pooled.shape
  N = w.shape[0]
  return pl.pallas_call(
      _linear_body,
      grid=(B // bm,),
      in_specs=[
          pl.BlockSpec((N, D), lambda i: (0, 0)),
          pl.BlockSpec((bm, D), lambda i: (i, 0)),
          pl.BlockSpec((N, 1), lambda i: (0, 0)),
      ],
      out_specs=pl.BlockSpec((N, bm), lambda i: (0, i)),
      out_shape=jax.ShapeDtypeStruct((N, B), jnp.float32),
  )(w, pooled, bias_col)


def kernel(inputs, embed_table, fc_w, fc_b):
  B, CTX = inputs.shape
  V, D = embed_table.shape
  N = fc_w.shape[0]
  b_per_w = B // _NW

  v = inputs.astype(jnp.int32)
  idx = 2 * ((v // (2 * _VB)) * _VB + (v % _VB)) + ((v // _VB) % 2)
  idx = jnp.transpose(idx.reshape(_NW, b_per_w, CTX), (0, 2, 1))
  table_lin = _transpose_detile(jnp.transpose(embed_table))
  pooled = _make_gather_pool(B, CTX, D, b_per_w)(idx, table_lin)
  logits_t = _linear(pooled, fc_w, fc_b.reshape(N, 1), bm=512)
  return jnp.transpose(logits_t)


# speedup vs baseline: 2.7221x; 1.3107x over previous
import functools

import jax
import jax.numpy as jnp
from jax import lax
from jax.experimental import pallas as pl
from jax.experimental.pallas import tpu as pltpu
from jax.experimental.pallas import tpu_sc as plsc

_NUM_CORES = 2
_NUM_SUBCORES = 16
_NW = _NUM_CORES * _NUM_SUBCORES
_LANES = 16


_VB = 4096


def _tr_body(x_ref, o_ref):
  x = x_ref[...]
  xc = jnp.concatenate([x[:, :_VB], x[:, _VB:]], axis=0)
  o_ref[...] = xc.T


def _transpose_detile(table):
  D, V = table.shape
  grid = -(-V // (2 * _VB))
  out = pl.pallas_call(
      _tr_body,
      grid=(grid,),
      in_specs=[pl.BlockSpec((D, 2 * _VB), lambda i: (0, i))],
      out_specs=pl.BlockSpec((_VB, 2 * D), lambda i: (i, 0)),
      out_shape=jax.ShapeDtypeStruct((grid * _VB, 2 * D), jnp.float32),
  )(table)
  return out.reshape(2 * grid * _VB, D)


def _make_gather_pool(B, CTX, D, b_per_w):
  mesh = plsc.VectorSubcoreMesh(
      core_axis_name="c", subcore_axis_name="s", num_cores=_NUM_CORES,
      num_subcores=_NUM_SUBCORES)

  @functools.partial(
      pl.kernel,
      mesh=mesh,
      compiler_params=pltpu.CompilerParams(use_tc_tiling_on_sc=False),
      out_type=jax.ShapeDtypeStruct((B, D), jnp.float32),
      scratch_types=[
          pltpu.VMEM((CTX, b_per_w), jnp.int32),
          pltpu.VMEM((b_per_w, D), jnp.float32),
          pltpu.SemaphoreType.DMA,
      ],
  )
  def gather_pool(idx_hbm, table_hbm, out_hbm, idx_t, acc_v, sem):
    wid = lax.axis_index("s") * _NUM_CORES + lax.axis_index("c")
    base = wid * b_per_w
    pltpu.sync_copy(idx_hbm.at[wid], idx_t)

    pltpu.async_copy(table_hbm.at[idx_t.at[0]], acc_v, sem).wait()

    def fire(j, carry):
      pltpu.async_copy(table_hbm.at[idx_t.at[j]], acc_v, sem, add=True)
      return carry

    lax.fori_loop(1, CTX, fire, 0)

    def drain(j, carry):
      pltpu.make_async_copy(table_hbm.at[idx_t.at[0]], acc_v, sem).wait()
      return carry

    lax.fori_loop(1, CTX, drain, 0)

    pltpu.sync_copy(acc_v, out_hbm.at[pl.ds(base, b_per_w)])

  return gather_pool


def _linear_body(w_ref, x_ref, b_ref, o_ref):
  o_ref[...] = (
      lax.dot_general(
          w_ref[...], x_ref[...], (((1,), (1,)), ((), ())),
          preferred_element_type=jnp.float32)
      + b_ref[...]
  )


def _linear(pooled, w, bias_col, bm):
  B, D = pooled.shape
  N = w.shape[0]
  return pl.pallas_call(
      _linear_body,
      grid=(B // bm,),
      in_specs=[
          pl.BlockSpec((N, D), lambda i: (0, 0)),
          pl.BlockSpec((bm, D), lambda i: (i, 0)),
          pl.BlockSpec((N, 1), lambda i: (0, 0)),
      ],
      out_specs=pl.BlockSpec((N, bm), lambda i: (0, i)),
      out_shape=jax.ShapeDtypeStruct((N, B), jnp.float32),
  )(w, pooled, bias_col)


def kernel(inputs, embed_table, fc_w, fc_b):
  B, CTX = inputs.shape
  V, D = embed_table.shape
  N = fc_w.shape[0]
  b_per_w = B // _NW

  v = inputs.astype(jnp.int32)
  idx = 2 * ((v // (2 * _VB)) * _VB + (v % _VB)) + ((v // _VB) % 2)
  idx = jnp.transpose(idx.reshape(_NW, b_per_w, CTX), (0, 2, 1))
  table_lin = _transpose_detile(jnp.transpose(embed_table))
  pooled = _make_gather_pool(B, CTX, D, b_per_w)(idx, table_lin)
  logits_t = _linear(pooled, fc_w, fc_b.reshape(N, 1), bm=512)
  return jnp.transpose(logits_t)


# speedup vs baseline: 3.0507x; 1.1207x over previous
import functools

import jax
import jax.numpy as jnp
from jax import lax
from jax.experimental import pallas as pl
from jax.experimental.pallas import tpu as pltpu
from jax.experimental.pallas import tpu_sc as plsc

_NUM_CORES = 2
_NUM_SUBCORES = 16
_NW = _NUM_CORES * _NUM_SUBCORES
_LANES = 16


_VB = 8192


def _tr_body(x_ref, o_ref):
  x = x_ref[...]
  xc = jnp.concatenate([x[:, :_VB], x[:, _VB:]], axis=0)
  o_ref[...] = xc.T


def _transpose_detile(table):
  D, V = table.shape
  grid = -(-V // (2 * _VB))
  out = pl.pallas_call(
      _tr_body,
      grid=(grid,),
      in_specs=[pl.BlockSpec((D, 2 * _VB), lambda i: (0, i))],
      out_specs=pl.BlockSpec((_VB, 2 * D), lambda i: (i, 0)),
      out_shape=jax.ShapeDtypeStruct((grid * _VB, 2 * D), jnp.float32),
  )(table)
  return out.reshape(2 * grid * _VB, D)


def _make_gather_pool(B, CTX, D, b_per_w):
  mesh = plsc.VectorSubcoreMesh(
      core_axis_name="c", subcore_axis_name="s", num_cores=_NUM_CORES,
      num_subcores=_NUM_SUBCORES)

  @functools.partial(
      pl.kernel,
      mesh=mesh,
      compiler_params=pltpu.CompilerParams(use_tc_tiling_on_sc=False),
      out_type=jax.ShapeDtypeStruct((B, D), jnp.float32),
      scratch_types=[
          pltpu.VMEM((CTX, b_per_w), jnp.int32),
          pltpu.VMEM((b_per_w, D), jnp.float32),
          pltpu.SemaphoreType.DMA,
      ],
  )
  def gather_pool(idx_hbm, table_hbm, out_hbm, idx_t, acc_v, sem):
    wid = lax.axis_index("s") * _NUM_CORES + lax.axis_index("c")
    base = wid * b_per_w
    pltpu.sync_copy(idx_hbm.at[wid], idx_t)

    pltpu.async_copy(table_hbm.at[idx_t.at[0]], acc_v, sem).wait()

    def fire(j, carry):
      pltpu.async_copy(table_hbm.at[idx_t.at[j]], acc_v, sem, add=True)
      return carry

    lax.fori_loop(1, CTX, fire, 0)

    def drain(j, carry):
      pltpu.make_async_copy(table_hbm.at[idx_t.at[0]], acc_v, sem).wait()
      return carry

    lax.fori_loop(1, CTX, drain, 0)

    pltpu.sync_copy(acc_v, out_hbm.at[pl.ds(base, b_per_w)])

  return gather_pool


def _linear_body(w_ref, x_ref, b_ref, o_ref):
  o_ref[...] = (
      lax.dot_general(
          w_ref[...], x_ref[...], (((1,), (1,)), ((), ())),
          preferred_element_type=jnp.float32)
      + b_ref[...]
  )


def _linear(pooled, w, bias_col, bm):
  B, D = pooled.shape
  N = w.shape[0]
  return pl.pallas_call(
      _linear_body,
      grid=(B // bm,),
      in_specs=[
          pl.BlockSpec((N, D), lambda i: (0, 0)),
          pl.BlockSpec((bm, D), lambda i: (i, 0)),
          pl.BlockSpec((N, 1), lambda i: (0, 0)),
      ],
      out_specs=pl.BlockSpec((N, bm), lambda i: (0, i)),
      out_shape=jax.ShapeDtypeStruct((N, B), jnp.float32),
  )(w, pooled, bias_col)


def kernel(inputs, embed_table, fc_w, fc_b):
  B, CTX = inputs.shape
  V, D = embed_table.shape
  N = fc_w.shape[0]
  b_per_w = B // _NW

  v = inputs.astype(jnp.int32)
  idx = 2 * ((v // (2 * _VB)) * _VB + (v % _VB)) + ((v // _VB) % 2)
  idx = jnp.transpose(idx.reshape(_NW, b_per_w, CTX), (0, 2, 1))
  table_lin = _transpose_detile(jnp.transpose(embed_table))
  pooled = _make_gather_pool(B, CTX, D, b_per_w)(idx, table_lin)
  logits_t = _linear(pooled, fc_w, fc_b.reshape(N, 1), bm=512)
  return jnp.transpose(logits_t)


# speedup vs baseline: 3.1195x; 1.0226x over previous
import functools

import jax
import jax.numpy as jnp
from jax import lax
from jax.experimental import pallas as pl
from jax.experimental.pallas import tpu as pltpu
from jax.experimental.pallas import tpu_sc as plsc

_NUM_CORES = 2
_NUM_SUBCORES = 16
_NW = _NUM_CORES * _NUM_SUBCORES
_LANES = 16


_VB = 16384


def _tr_body(x_ref, o_ref):
  x = x_ref[...]
  xc = jnp.concatenate([x[:, :_VB], x[:, _VB:]], axis=0)
  o_ref[...] = xc.T


def _transpose_detile(table):
  D, V = table.shape
  grid = -(-V // (2 * _VB))
  out = pl.pallas_call(
      _tr_body,
      grid=(grid,),
      in_specs=[pl.BlockSpec((D, 2 * _VB), lambda i: (0, i))],
      out_specs=pl.BlockSpec((_VB, 2 * D), lambda i: (i, 0)),
      out_shape=jax.ShapeDtypeStruct((grid * _VB, 2 * D), jnp.float32),
  )(table)
  return out.reshape(2 * grid * _VB, D)


def _make_gather_pool(B, CTX, D, b_per_w):
  mesh = plsc.VectorSubcoreMesh(
      core_axis_name="c", subcore_axis_name="s", num_cores=_NUM_CORES,
      num_subcores=_NUM_SUBCORES)

  @functools.partial(
      pl.kernel,
      mesh=mesh,
      compiler_params=pltpu.CompilerParams(use_tc_tiling_on_sc=False),
      out_type=jax.ShapeDtypeStruct((B, D), jnp.float32),
      scratch_types=[
          pltpu.VMEM((CTX, b_per_w), jnp.int32),
          pltpu.VMEM((b_per_w, D), jnp.float32),
          pltpu.SemaphoreType.DMA,
      ],
  )
  def gather_pool(idx_hbm, table_hbm, out_hbm, idx_t, acc_v, sem):
    wid = lax.axis_index("s") * _NUM_CORES + lax.axis_index("c")
    base = wid * b_per_w
    pltpu.sync_copy(idx_hbm.at[wid], idx_t)

    pltpu.async_copy(table_hbm.at[idx_t.at[0]], acc_v, sem).wait()

    def fire(j, carry):
      pltpu.async_copy(table_hbm.at[idx_t.at[j]], acc_v, sem, add=True)
      return carry

    lax.fori_loop(1, CTX, fire, 0)

    def drain(j, carry):
      pltpu.make_async_copy(table_hbm.at[idx_t.at[0]], acc_v, sem).wait()
      return carry

    lax.fori_loop(1, CTX, drain, 0)

    pltpu.sync_copy(acc_v, out_hbm.at[pl.ds(base, b_per_w)])

  return gather_pool


def _linear_body(w_ref, x_ref, b_ref, o_ref):
  o_ref[...] = (
      lax.dot_general(
          w_ref[...], x_ref[...], (((1,), (1,)), ((), ())),
          preferred_element_type=jnp.float32)
      + b_ref[...]
  )


def _linear(pooled, w, bias_col, bm):
  B, D = pooled.shape
  N = w.shape[0]
  return pl.pallas_call(
      _linear_body,
      grid=(B // bm,),
      in_specs=[
          pl.BlockSpec((N, D), lambda i: (0, 0)),
          pl.BlockSpec((bm, D), lambda i: (i, 0)),
          pl.BlockSpec((N, 1), lambda i: (0, 0)),
      ],
      out_specs=pl.BlockSpec((N, bm), lambda i: (0, i)),
      out_shape=jax.ShapeDtypeStruct((N, B), jnp.float32),
  )(w, pooled, bias_col)


def kernel(inputs, embed_table, fc_w, fc_b):
  B, CTX = inputs.shape
  V, D = embed_table.shape
  N = fc_w.shape[0]
  b_per_w = B // _NW

  v = inputs.astype(jnp.int32)
  idx = 2 * ((v // (2 * _VB)) * _VB + (v % _VB)) + ((v // _VB) % 2)
  idx = jnp.transpose(idx.reshape(_NW, b_per_w, CTX), (0, 2, 1))
  table_lin = _transpose_detile(jnp.transpose(embed_table))
  pooled = _make_gather_pool(B, CTX, D, b_per_w)(idx, table_lin)
  logits_t = _linear(pooled, fc_w, fc_b.reshape(N, 1), bm=512)
  return jnp.transpose(logits_t)
